# Initial kernel scaffold; baseline (speedup 1.0000x reference)
import functools

import jax
import jax.numpy as jnp
from jax import lax
from jax.experimental import pallas as pl
from jax.experimental.pallas import tpu as pltpu
from jax.experimental.pallas import tpu_sc as plsc

N = 10000
E = 320000
D = 128
NC = 2
NS = 16
NW = NC * NS
L = 16
DH = D // NC

ROWS_PER_TILE = N // NS
EPT = E // NS
EPW = E // NW
CH = 80

_mesh = plsc.VectorSubcoreMesh(
    core_axis_name="c", subcore_axis_name="s", num_cores=NC, num_subcores=NS)


@functools.partial(
    pl.kernel,
    out_type=jax.ShapeDtypeStruct((NC, N), jnp.float32),
    mesh=_mesh,
    scratch_types=[
        pltpu.VMEM((EPW,), jnp.int32),
        pltpu.VMEM((N,), jnp.float32),
        pltpu.VMEM_SHARED((N,), jnp.float32),
    ],
)
def _deg_kernel(dst_hbm, deg_hbm, idx_v, deg_v, shared):
    c = lax.axis_index("c")
    s = lax.axis_index("s")
    wid = s * NC + c

    fill = jnp.where(wid == 0, 1.0, 0.0).astype(jnp.float32)
    val = jnp.full((L,), 1.0, jnp.float32) * fill

    def zbody(i, _):
        deg_v[pl.ds(i * L, L)] = val
        return 0

    lax.fori_loop(0, N // L, zbody, 0)

    pltpu.sync_copy(dst_hbm.at[pl.ds(wid * EPW, EPW)], idx_v)
    ones = jnp.full((L,), 1.0, jnp.float32)

    def body(i, _):
        idx = idx_v[pl.ds(i * L, L)]
        plsc.addupdate_scatter(deg_v, [idx], ones)
        return 0

    lax.fori_loop(0, EPW // L, body, 0)

    @pl.when(s == 0)
    def _():
        pltpu.sync_copy(deg_v, shared)

    plsc.subcore_barrier()

    @pl.when(s != 0)
    def _():
        pltpu.sync_copy(deg_v, shared, add=True)

    plsc.subcore_barrier()

    @pl.when(s < 10)
    def _():
        seg = pl.ds(s * 1000, 1000)
        pltpu.sync_copy(shared.at[seg], deg_v.at[pl.ds(0, 1000)])
        pltpu.sync_copy(deg_v.at[pl.ds(0, 1000)], deg_hbm.at[c].at[seg])


def _mm_body(x_ref, w_ref, deg_ref, g_ref, dinv_ref):
    h = jnp.dot(x_ref[...], w_ref[...], preferred_element_type=jnp.float32)
    deg = deg_ref[0] + deg_ref[1]
    dinv = lax.rsqrt(deg)
    g = h * dinv
    g_ref[0] = g[:, :DH]
    g_ref[1] = g[:, DH:]
    dinv_ref[...] = dinv


def _matmul_scale(x, W, deg2):
    bm = 1000
    return pl.pallas_call(
        _mm_body,
        grid=(N // bm,),
        in_specs=[
            pl.BlockSpec((bm, D), lambda i: (i, 0)),
            pl.BlockSpec((D, D), lambda i: (0, 0)),
            pl.BlockSpec((NC, bm, 1), lambda i: (0, i, 0)),
        ],
        out_specs=[
            pl.BlockSpec((NC, bm, DH), lambda i: (0, i, 0)),
            pl.BlockSpec((bm, 1), lambda i: (i, 0)),
        ],
        out_shape=[
            jax.ShapeDtypeStruct((NC, N, DH), jnp.float32),
            jax.ShapeDtypeStruct((N, 1), jnp.float32),
        ],
    )(x, W, deg2.reshape(NC, N, 1))


@functools.partial(
    pl.kernel,
    out_type=jax.ShapeDtypeStruct((N, D), jnp.float32),
    mesh=_mesh,
    scratch_types=[
        pltpu.VMEM((CH,), jnp.int32),
        pltpu.VMEM((CH,), jnp.int32),
        pltpu.VMEM((CH, DH), jnp.float32),
        pltpu.VMEM((ROWS_PER_TILE, DH), jnp.float32),
        pltpu.VMEM((ROWS_PER_TILE,), jnp.float32),
        pltpu.VMEM((DH,), jnp.float32),
        pltpu.VMEM_SHARED((N, DH), jnp.float32),
        pltpu.SemaphoreType.DMA,
    ],
)
def _edge_kernel(g2_hbm, src_hbm, dst_hbm, dinv_hbm, b_hbm, out_hbm,
                 src_v, dst_v, rows_v, stage_v, dinv_v, bias_v, shared, sem):
    c = lax.axis_index("c")
    s = lax.axis_index("s")
    r0 = s * ROWS_PER_TILE

    pltpu.sync_copy(g2_hbm.at[c].at[pl.ds(r0, ROWS_PER_TILE), :], stage_v)
    pltpu.sync_copy(stage_v, shared.at[pl.ds(r0, ROWS_PER_TILE), :])
    plsc.subcore_barrier()

    base = s * EPT

    def body(i, _):
        off = base + i * CH
        pltpu.sync_copy(src_hbm.at[pl.ds(off, CH)], src_v)
        pltpu.sync_copy(dst_hbm.at[pl.ds(off, CH)], dst_v)
        pltpu.async_copy(g2_hbm.at[c].at[src_v], rows_v, sem).wait()
        pltpu.sync_copy(rows_v, shared.at[dst_v], add=True)
        return 0

    lax.fori_loop(0, EPT // CH, body, 0)
    plsc.subcore_barrier()

    pltpu.sync_copy(shared.at[pl.ds(r0, ROWS_PER_TILE), :], stage_v)
    pltpu.sync_copy(dinv_hbm.at[s], dinv_v)
    pltpu.sync_copy(b_hbm.at[c], bias_v)

    def row_body(r, _):
        dv = plsc.load_gather(dinv_v, [jnp.full((L,), r, jnp.int32)])
        for k in range(DH // L):
            seg = pl.ds(k * L, L)
            v = stage_v[r, seg] * dv + bias_v[seg]
            stage_v[r, seg] = jnp.maximum(v, 0.0)
        return 0

    lax.fori_loop(0, ROWS_PER_TILE, row_body, 0)
    pltpu.sync_copy(stage_v,
                    out_hbm.at[pl.ds(r0, ROWS_PER_TILE), pl.ds(c * DH, DH)])


def kernel(x, edge_index, W, b):
    src = edge_index[0].astype(jnp.int32)
    dst = edge_index[1].astype(jnp.int32)
    deg2 = _deg_kernel(dst)
    g2, dinv = _matmul_scale(x, W, deg2)
    out = _edge_kernel(g2, src, dst,
                       dinv.reshape(NS, ROWS_PER_TILE),
                       b.reshape(NC, DH))
    return out


# speedup vs baseline: 19.6033x; 19.6033x over previous
import functools

import jax
import jax.numpy as jnp
from jax import lax
from jax.experimental import pallas as pl
from jax.experimental.pallas import tpu as pltpu
from jax.experimental.pallas import tpu_sc as plsc

N = 10000
E = 320000
D = 128
NC = 2
NS = 16
NW = NC * NS
L = 16
CH = 80

EPW = E // NW
RT = 640
RT_LAST = N - 15 * RT

_mesh = plsc.VectorSubcoreMesh(
    core_axis_name="c", subcore_axis_name="s", num_cores=NC, num_subcores=NS)


@functools.partial(
    pl.kernel,
    out_type=jax.ShapeDtypeStruct((NC, N), jnp.float32),
    mesh=_mesh,
    scratch_types=[
        pltpu.VMEM((EPW // CH, CH), jnp.int32),
        pltpu.VMEM((N,), jnp.float32),
        pltpu.VMEM((CH,), jnp.float32),
        pltpu.VMEM_SHARED((N,), jnp.float32),
    ],
)
def _deg_kernel(dst3_hbm, init_hbm, ones_hbm, deg_hbm, idx_v, nbuf_v, ones_v,
                shared):
    c = lax.axis_index("c")
    s = lax.axis_index("s")
    wid = s * NC + c

    @pl.when(s == 0)
    def _():
        pltpu.sync_copy(init_hbm.at[c], nbuf_v)
        pltpu.sync_copy(nbuf_v, shared)

    pltpu.sync_copy(ones_hbm, ones_v)
    pltpu.sync_copy(dst3_hbm.at[wid], idx_v)
    plsc.subcore_barrier()

    def body(i, _):
        pltpu.sync_copy(ones_v, shared.at[idx_v.at[i]], add=True)
        return 0

    lax.fori_loop(0, EPW // CH, body, 0)
    plsc.subcore_barrier()

    @pl.when(s == 0)
    def _():
        pltpu.sync_copy(shared, nbuf_v)
        pltpu.sync_copy(nbuf_v, deg_hbm.at[c])


def _mm_body(x_ref, w_ref, deg_ref, g_ref):
    h = jnp.dot(x_ref[...], w_ref[...], preferred_element_type=jnp.float32)
    deg = deg_ref[0] + deg_ref[1]
    g_ref[...] = h * lax.rsqrt(deg)


def _matmul_scale(x, W, deg2):
    bm = 1000
    return pl.pallas_call(
        _mm_body,
        grid=(N // bm,),
        in_specs=[
            pl.BlockSpec((bm, D), lambda i: (i, 0)),
            pl.BlockSpec((D, D), lambda i: (0, 0)),
            pl.BlockSpec((NC, bm, 1), lambda i: (0, i, 0)),
        ],
        out_specs=pl.BlockSpec((bm, D), lambda i: (i, 0)),
        out_shape=jax.ShapeDtypeStruct((N, D), jnp.float32),
    )(x, W, deg2)


@functools.partial(
    pl.kernel,
    out_type=jax.ShapeDtypeStruct((NC, N, D), jnp.float32),
    mesh=_mesh,
    scratch_types=[
        pltpu.VMEM((CH,), jnp.int32),
        pltpu.VMEM((CH,), jnp.int32),
        pltpu.VMEM((CH, D), jnp.float32),
        pltpu.VMEM((CH, D), jnp.float32),
        pltpu.VMEM_SHARED((N, D), jnp.float32),
        pltpu.SemaphoreType.DMA,
    ],
)
def _edge_kernel(g_hbm, src_hbm, dst_hbm, zero_hbm, agg_hbm,
                 src_v, dst_v, rows_v, stage_v, shared, sem):
    c = lax.axis_index("c")
    s = lax.axis_index("s")
    r0 = pl.multiple_of(s * RT, 8)
    nch = jnp.where(s == NS - 1, RT_LAST // CH, RT // CH)

    @pl.when(c == 1)
    def _():
        pltpu.sync_copy(zero_hbm, stage_v)

    def init_body(j, _):
        rows = pl.ds(pl.multiple_of(r0 + j * CH, 8), CH)

        @pl.when(c == 0)
        def _():
            pltpu.sync_copy(g_hbm.at[rows, :], stage_v)
        pltpu.sync_copy(stage_v, shared.at[rows, :])
        return 0

    lax.fori_loop(0, nch, init_body, 0)
    plsc.subcore_barrier()

    wid = s * NC + c
    base = wid * EPW

    def body(i, _):
        off = base + i * CH
        pltpu.sync_copy(src_hbm.at[pl.ds(off, CH)], src_v)
        pltpu.sync_copy(dst_hbm.at[pl.ds(off, CH)], dst_v)
        pltpu.async_copy(g_hbm.at[src_v], rows_v, sem).wait()
        pltpu.sync_copy(rows_v, shared.at[dst_v], add=True)
        return 0

    lax.fori_loop(0, EPW // CH, body, 0)
    plsc.subcore_barrier()

    def out_body(j, _):
        rows = pl.ds(pl.multiple_of(r0 + j * CH, 8), CH)
        pltpu.sync_copy(shared.at[rows, :], stage_v)
        pltpu.sync_copy(stage_v, agg_hbm.at[c].at[rows, :])
        return 0

    lax.fori_loop(0, nch, out_body, 0)


def _out_body(agg_ref, deg_ref, b_ref, o_ref):
    deg = deg_ref[0] + deg_ref[1]
    dinv = lax.rsqrt(deg)
    acc = agg_ref[0] + agg_ref[1]
    o_ref[...] = jnp.maximum(acc * dinv + b_ref[...], 0.0)


def _combine(agg, deg2, b):
    bm = 1000
    return pl.pallas_call(
        _out_body,
        grid=(N // bm,),
        in_specs=[
            pl.BlockSpec((NC, bm, D), lambda i: (0, i, 0)),
            pl.BlockSpec((NC, bm, 1), lambda i: (0, i, 0)),
            pl.BlockSpec((1, D), lambda i: (0, 0)),
        ],
        out_specs=pl.BlockSpec((bm, D), lambda i: (i, 0)),
        out_shape=jax.ShapeDtypeStruct((N, D), jnp.float32),
    )(agg, deg2, b.reshape(1, D))


def kernel(x, edge_index, W, b):
    src = edge_index[0].astype(jnp.int32)
    dst = edge_index[1].astype(jnp.int32)
    init = jnp.stack([jnp.ones((N,), jnp.float32),
                      jnp.zeros((N,), jnp.float32)])
    ones = jnp.ones((CH,), jnp.float32)
    zero = jnp.zeros((CH, D), jnp.float32)
    deg2 = _deg_kernel(dst.reshape(NW, EPW // CH, CH), init, ones)
    g = _matmul_scale(x, W, deg2.reshape(NC, N, 1))
    agg = _edge_kernel(g, src, dst, zero)
    return _combine(agg, deg2.reshape(NC, N, 1), b)


# speedup vs baseline: 35.7601x; 1.8242x over previous
import functools

import jax
import jax.numpy as jnp
from jax import lax
from jax.experimental import pallas as pl
from jax.experimental.pallas import tpu as pltpu
from jax.experimental.pallas import tpu_sc as plsc

N = 10000
E = 320000
D = 128
NC = 2
NS = 16
NW = NC * NS
L = 16
CH = 80

EPW = E // NW
SB = 25
RT = 640
RT_LAST = N - 15 * RT

_mesh = plsc.VectorSubcoreMesh(
    core_axis_name="c", subcore_axis_name="s", num_cores=NC, num_subcores=NS)


@functools.partial(
    pl.kernel,
    out_type=jax.ShapeDtypeStruct((NC, N), jnp.float32),
    mesh=_mesh,
    scratch_types=[
        pltpu.VMEM((EPW // CH, CH), jnp.int32),
        pltpu.VMEM((N,), jnp.float32),
        pltpu.VMEM((CH,), jnp.float32),
        pltpu.VMEM_SHARED((N,), jnp.float32),
    ],
)
def _deg_kernel(dst3_hbm, init_hbm, ones_hbm, deg_hbm, idx_v, nbuf_v, ones_v,
                shared):
    c = lax.axis_index("c")
    s = lax.axis_index("s")
    wid = s * NC + c

    @pl.when(s == 0)
    def _():
        pltpu.sync_copy(init_hbm.at[c], nbuf_v)
        pltpu.sync_copy(nbuf_v, shared)

    pltpu.sync_copy(ones_hbm, ones_v)
    pltpu.sync_copy(dst3_hbm.at[wid], idx_v)
    plsc.subcore_barrier()

    def body(i, _):
        pltpu.sync_copy(ones_v, shared.at[idx_v.at[i]], add=True)
        return 0

    lax.fori_loop(0, EPW // CH, body, 0)
    plsc.subcore_barrier()

    @pl.when(s == 0)
    def _():
        pltpu.sync_copy(shared, nbuf_v)
        pltpu.sync_copy(nbuf_v, deg_hbm.at[c])


def _mm_body(x_ref, w_ref, deg_ref, g_ref):
    h = jnp.dot(x_ref[...], w_ref[...], preferred_element_type=jnp.float32)
    deg = deg_ref[0] + deg_ref[1]
    g_ref[...] = h * lax.rsqrt(deg)


def _matmul_scale(x, W, deg2):
    bm = 1000
    return pl.pallas_call(
        _mm_body,
        grid=(N // bm,),
        in_specs=[
            pl.BlockSpec((bm, D), lambda i: (i, 0)),
            pl.BlockSpec((D, D), lambda i: (0, 0)),
            pl.BlockSpec((NC, bm, 1), lambda i: (0, i, 0)),
        ],
        out_specs=pl.BlockSpec((bm, D), lambda i: (i, 0)),
        out_shape=jax.ShapeDtypeStruct((N, D), jnp.float32),
    )(x, W, deg2)


@functools.partial(
    pl.kernel,
    out_type=jax.ShapeDtypeStruct((NC, N, D), jnp.float32),
    mesh=_mesh,
    scratch_types=[
        pltpu.VMEM((SB, CH), jnp.int32),
        pltpu.VMEM((SB, CH), jnp.int32),
        pltpu.VMEM((CH, D), jnp.float32),
        pltpu.VMEM((CH, D), jnp.float32),
        pltpu.VMEM_SHARED((N, D), jnp.float32),
        pltpu.SemaphoreType.DMA,
        pltpu.SemaphoreType.DMA,
    ],
)
def _edge_kernel(g_hbm, src4_hbm, dst4_hbm, zero_hbm, agg_hbm,
                 sidx_v, didx_v, rows0_v, rows1_v, shared, sem0, sem1):
    c = lax.axis_index("c")
    s = lax.axis_index("s")
    r0 = pl.multiple_of(s * RT, 8)
    nch = jnp.where(s == NS - 1, RT_LAST // CH, RT // CH)

    @pl.when(c == 1)
    def _():
        pltpu.sync_copy(zero_hbm, rows0_v)

    def init_body(j, _):
        rows = pl.ds(pl.multiple_of(r0 + j * CH, 8), CH)

        @pl.when(c == 0)
        def _():
            pltpu.sync_copy(g_hbm.at[rows, :], rows0_v)
        pltpu.sync_copy(rows0_v, shared.at[rows, :])
        return 0

    lax.fori_loop(0, nch, init_body, 0)

    wid = s * NC + c
    plsc.subcore_barrier()

    def gather(ch, buf, sem):
        pltpu.async_copy(g_hbm.at[sidx_v.at[ch]], buf, sem)

    def gwait(ch, buf, sem):
        pltpu.make_async_copy(g_hbm.at[sidx_v.at[ch]], buf, sem).wait()

    def scat(ch, buf):
        pltpu.sync_copy(buf, shared.at[didx_v.at[ch]], add=True)

    def sb_body(sb, _):
        pltpu.sync_copy(src4_hbm.at[wid].at[sb], sidx_v)
        pltpu.sync_copy(dst4_hbm.at[wid].at[sb], didx_v)
        gather(0, rows0_v, sem0)

        def body(j, _):
            gather(2 * j + 1, rows1_v, sem1)
            gwait(2 * j, rows0_v, sem0)
            scat(2 * j, rows0_v)
            gather(2 * j + 2, rows0_v, sem0)
            gwait(2 * j + 1, rows1_v, sem1)
            scat(2 * j + 1, rows1_v)
            return 0

        lax.fori_loop(0, (SB - 1) // 2, body, 0)
        gwait(SB - 1, rows0_v, sem0)
        scat(SB - 1, rows0_v)
        return 0

    lax.fori_loop(0, EPW // (SB * CH), sb_body, 0)
    plsc.subcore_barrier()

    def out_body(j, _):
        rows = pl.ds(pl.multiple_of(r0 + j * CH, 8), CH)
        pltpu.sync_copy(shared.at[rows, :], rows0_v)
        pltpu.sync_copy(rows0_v, agg_hbm.at[c].at[rows, :])
        return 0

    lax.fori_loop(0, nch, out_body, 0)


def _out_body(agg_ref, deg_ref, b_ref, o_ref):
    deg = deg_ref[0] + deg_ref[1]
    dinv = lax.rsqrt(deg)
    acc = agg_ref[0] + agg_ref[1]
    o_ref[...] = jnp.maximum(acc * dinv + b_ref[...], 0.0)


def _combine(agg, deg2, b):
    bm = 1000
    return pl.pallas_call(
        _out_body,
        grid=(N // bm,),
        in_specs=[
            pl.BlockSpec((NC, bm, D), lambda i: (0, i, 0)),
            pl.BlockSpec((NC, bm, 1), lambda i: (0, i, 0)),
            pl.BlockSpec((1, D), lambda i: (0, 0)),
        ],
        out_specs=pl.BlockSpec((bm, D), lambda i: (i, 0)),
        out_shape=jax.ShapeDtypeStruct((N, D), jnp.float32),
    )(agg, deg2, b.reshape(1, D))


def kernel(x, edge_index, W, b):
    src = edge_index[0].astype(jnp.int32)
    dst = edge_index[1].astype(jnp.int32)
    init = jnp.stack([jnp.ones((N,), jnp.float32),
                      jnp.zeros((N,), jnp.float32)])
    ones = jnp.ones((CH,), jnp.float32)
    zero = jnp.zeros((CH, D), jnp.float32)
    deg2 = _deg_kernel(dst.reshape(NW, EPW // CH, CH), init, ones)
    g = _matmul_scale(x, W, deg2.reshape(NC, N, 1))
    agg = _edge_kernel(g, src.reshape(NW, EPW // (SB * CH), SB, CH),
                       dst.reshape(NW, EPW // (SB * CH), SB, CH), zero)
    return _combine(agg, deg2.reshape(NC, N, 1), b)
